# baseline (speedup 1.0000x reference)
import functools

import jax
import jax.numpy as jnp
from jax import lax
from jax.experimental import pallas as pl
from jax.experimental.pallas import tpu as pltpu
from jax.experimental.pallas import tpu_sc as plsc

K = 9
EPS = 1e-5
LANES = 16
NTILES = 16
NCORES = 2


def _gemm_body(x_ref, w_ref, o_ref):
    o_ref[...] = jnp.dot(x_ref[...], w_ref[...],
                         preferred_element_type=jnp.float32)


def _gemm(x, wcat, blk):
    n, in_c = x.shape
    kc = wcat.shape[1]
    return pl.pallas_call(
        _gemm_body,
        grid=(n // blk,),
        in_specs=[pl.BlockSpec((blk, in_c), lambda i: (i, 0)),
                  pl.BlockSpec((in_c, kc), lambda i: (0, 0))],
        out_specs=pl.BlockSpec((blk, kc), lambda i: (i, 0)),
        out_shape=jax.ShapeDtypeStruct((n, kc), jnp.float32),
    )(x, wcat)


def _stats_body(x_ref, o_ref):
    @pl.when(pl.program_id(0) == 0)
    def _():
        o_ref[...] = jnp.zeros_like(o_ref)

    x = x_ref[...]
    o_ref[...] += jnp.stack([jnp.sum(x, axis=0), jnp.sum(x * x, axis=0)])


def _stats(x_pad, n, blk):
    c = x_pad.shape[1]
    return pl.pallas_call(
        _stats_body,
        grid=(n // blk,),
        in_specs=[pl.BlockSpec((blk, c), lambda i: (i, 0))],
        out_specs=pl.BlockSpec((2, c), lambda i: (0, 0)),
        out_shape=jax.ShapeDtypeStruct((2, c), jnp.float32),
    )(x_pad)


def _apply_body(x_ref, sc_ref, sh_ref, skip_ref, o_ref):
    o_ref[...] = x_ref[...] * sc_ref[0, :] + sh_ref[0, :] + skip_ref[...]


def _bn_apply(x_pad, scale, shift, skip, blk):
    n, c = skip.shape
    return pl.pallas_call(
        _apply_body,
        grid=(n // blk,),
        in_specs=[pl.BlockSpec((blk, c), lambda i: (i, 0)),
                  pl.BlockSpec((1, c), lambda i: (0, 0)),
                  pl.BlockSpec((1, c), lambda i: (0, 0)),
                  pl.BlockSpec((blk, c), lambda i: (i, 0))],
        out_specs=pl.BlockSpec((blk, c), lambda i: (i, 0)),
        out_shape=jax.ShapeDtypeStruct((n, c), jnp.float32),
    )(x_pad, scale, shift, skip)


def _make_sc_scatter(n_src_rows, n_out, out_c, Pp, C, Wn, G, R):
    NCH = -(-n_out // C)
    NP = -(-NCH // NCORES)
    S = Pp // NTILES
    nwin = S // Wn
    NW = K * nwin
    assert NW % 2 == 0
    GV = 8
    ngrp = Wn // (LANES * GV)
    nblk = C // (NTILES * R)
    acc_rows = C + LANES
    n_out_pad = NCH * C

    mesh = plsc.VectorSubcoreMesh(core_axis_name="c", subcore_axis_name="s")

    @functools.partial(
        pl.kernel,
        out_type=jax.ShapeDtypeStruct((n_out_pad, out_c), jnp.float32),
        mesh=mesh,
        scratch_types=[
            pltpu.VMEM((2, Wn), jnp.int32),
            pltpu.VMEM((2, Wn), jnp.int32),
            pltpu.VMEM((G,), jnp.int32),
            pltpu.VMEM((G,), jnp.int32),
            pltpu.VMEM((G, out_c), jnp.float32),
            pltpu.VMEM((R, out_c), jnp.float32),
            pltpu.VMEM_SHARED((acc_rows, out_c), jnp.float32),
            pltpu.SemaphoreType.DMA,
            pltpu.SemaphoreType.DMA,
            pltpu.SemaphoreType.DMA,
        ],
        compiler_params=pltpu.CompilerParams(needs_layout_passes=False,
                                             use_tc_tiling_on_sc=False),
    )
    def sck(y_hbm, c_hbm, z_hbm, out_hbm,
            bufA, bufB, src_st, dst_st, rows, zbuf, acc, sem, sem2, sem3):
        cid = lax.axis_index("c")
        sid = lax.axis_index("s")
        lane = lax.iota(jnp.int32, LANES)
        dummy_src = lane
        dummy_dst = C + lane
        uC = jnp.uint32(C)

        pltpu.sync_copy(z_hbm, zbuf)

        if True:
            def flush():
                copy = pltpu.async_copy(y_hbm.at[src_st], rows, sem3)
                copy.wait()
                pltpu.sync_copy(rows, acc.at[dst_st], add=True)

            def pass_body(p, _):
                chunk = NCORES * p + cid
                valid = chunk < NCH
                base = chunk * C

                @pl.when(valid)
                def _zero():
                    for b in range(nblk):
                        r0 = (sid * nblk + b) * R
                        pltpu.sync_copy(zbuf, acc.at[pl.ds(r0, R)])

                plsc.subcore_barrier()

                def slab(widx):
                    kk = widx // nwin
                    ww = widx % nwin
                    return kk * (NTILES * nwin) + sid * nwin + ww

                def process(buf, widx, off_v):
                    kk = widx // nwin

                    def gbody(gi, off_v):
                        off_s = jnp.max(off_v)
                        do_flush = off_s > G - GV * LANES

                        @pl.when(do_flush)
                        def _():
                            for t in range(GV):
                                fi = off_s + t * LANES + lane
                                fm = fi < G
                                plsc.store_scatter(src_st, [fi],
                                                   dummy_src, mask=fm)
                                plsc.store_scatter(dst_st, [fi],
                                                   dummy_dst, mask=fm)
                            flush()

                        off_v = jnp.where(do_flush,
                                          jnp.zeros_like(off_v), off_v)
                        for t in range(GV):
                            j = gi * GV + t
                            ov = buf[0, pl.ds(j * LANES, LANES)]
                            iv = buf[1, pl.ds(j * LANES, LANES)]
                            rel = ov - base
                            m = rel.astype(jnp.uint32) < uC
                            mi = m.astype(jnp.int32)
                            pos = off_v + plsc.cumsum(mi) - mi
                            src = iv * K + kk
                            plsc.store_scatter(dst_st, [pos], rel, mask=m)
                            plsc.store_scatter(src_st, [pos], src, mask=m)
                            off_v = off_v + \
                                plsc.all_reduce_population_count(m)
                        return off_v

                    return lax.fori_loop(0, ngrp, gbody, off_v)

                pltpu.async_copy(c_hbm.at[slab(0)], bufA, sem)
                pltpu.async_copy(c_hbm.at[slab(1)], bufB, sem2)

                def wpair_body(i, off_v):
                    w0 = 2 * i
                    pltpu.make_async_copy(c_hbm.at[slab(w0)],
                                          bufA, sem).wait()
                    off_v = process(bufA, w0, off_v)

                    @pl.when(w0 + 2 < NW)
                    def _():
                        pltpu.async_copy(c_hbm.at[slab(w0 + 2)], bufA, sem)

                    pltpu.make_async_copy(c_hbm.at[slab(w0 + 1)],
                                          bufB, sem2).wait()
                    off_v = process(bufB, w0 + 1, off_v)

                    @pl.when(w0 + 3 < NW)
                    def _():
                        pltpu.async_copy(c_hbm.at[slab(w0 + 3)], bufB, sem2)

                    return off_v

                off_v = lax.fori_loop(0, NW // 2, wpair_body,
                                      jnp.zeros((LANES,), jnp.int32))

                @pl.when(valid)
                def _final():
                    off_s = jnp.max(off_v)

                    def fill(i, _):
                        fi = off_s + i * LANES + lane
                        fm = fi < G
                        plsc.store_scatter(src_st, [fi], dummy_src, mask=fm)
                        plsc.store_scatter(dst_st, [fi], dummy_dst, mask=fm)
                        return _

                    lax.fori_loop(0, G // LANES, fill, jnp.int32(0))
                    flush()

                plsc.subcore_barrier()

                @pl.when(valid)
                def _writeout():
                    for b in range(nblk):
                        r0 = (sid * nblk + b) * R
                        pltpu.sync_copy(acc.at[pl.ds(r0, R)],
                                        rows.at[pl.ds(0, R)])
                        pltpu.sync_copy(rows.at[pl.ds(0, R)],
                                        out_hbm.at[pl.ds(base + r0, R)])

                plsc.subcore_barrier()
                return _

            lax.fori_loop(0, NP, pass_body, jnp.int32(0))

    return sck, n_out_pad


def _pad_pairs(idx, Pp, fill):
    pad = Pp - idx.shape[1]
    return jnp.pad(idx.astype(jnp.int32), ((0, 0), (0, pad)),
                   constant_values=fill)


def _stage(x, W, g, b, skip, in_idx, out_idx, cfg):
    n_out = skip.shape[0]
    in_c = x.shape[1]
    out_c = W.shape[-1]
    C, Wn, G, R, Pp, gemm_blk, ew_blk = cfg

    wcat = jnp.transpose(W, (1, 0, 2)).reshape(in_c, K * out_c)
    y = _gemm(x, wcat, gemm_blk).reshape(-1, out_c)

    sck, n_out_pad = _make_sc_scatter(y.shape[0], n_out, out_c, Pp, C, Wn, G, R)
    oi = _pad_pairs(out_idx, Pp, n_out_pad + C)
    ii = _pad_pairs(in_idx, Pp, 0)
    nwin = Pp // (NTILES * Wn)
    comb = jnp.stack([oi.reshape(K, NTILES, nwin, Wn),
                      ii.reshape(K, NTILES, nwin, Wn)], axis=3)
    comb = comb.reshape(K * NTILES * nwin, 2, Wn)
    zblk = jnp.zeros((R, out_c), jnp.float32)
    out_pre = sck(y, comb, zblk)

    st = _stats(out_pre, n_out, ew_blk)
    mean = st[0] / n_out
    var = st[1] / n_out - mean * mean
    scale = (g / jnp.sqrt(var + EPS)).reshape(1, out_c)
    shift = (b - mean * scale[0]).reshape(1, out_c)
    return _bn_apply(out_pre, scale, shift, skip, ew_blk)


def kernel(x_feat, skip3, skip2, skip1, W3, g3, b3, W2, g2, b2, W1, g1, b1,
           in_idx3, out_idx3, in_idx2, out_idx2, in_idx1, out_idx1):
    cfg3 = (1600, 2048, 1024, 100, 65536,  1000, 1000)
    cfg2 = (6400, 6656, 1024, 400, 212992, 1000, 1000)
    cfg1 = (12800, 6400, 1024, 800, 409600, 1000, 1000)
    x = _stage(x_feat, W3, g3, b3, skip3, in_idx3, out_idx3, cfg3)
    x = _stage(x, W2, g2, b2, skip2, in_idx2, out_idx2, cfg2)
    x = _stage(x, W1, g1, b1, skip1, in_idx1, out_idx1, cfg1)
    return x


# speedup vs baseline: 1.6246x; 1.6246x over previous
import functools

import jax
import jax.numpy as jnp
from jax import lax
from jax.experimental import pallas as pl
from jax.experimental.pallas import tpu as pltpu
from jax.experimental.pallas import tpu_sc as plsc

K = 9
EPS = 1e-5
LANES = 16
NTILES = 16
NCORES = 2


def _gemm_body(x_ref, w_ref, o_ref):
    o_ref[...] = jnp.dot(x_ref[...], w_ref[...],
                         preferred_element_type=jnp.float32)


def _gemm(x, wcat, blk):
    n, in_c = x.shape
    kc = wcat.shape[1]
    return pl.pallas_call(
        _gemm_body,
        grid=(n // blk,),
        in_specs=[pl.BlockSpec((blk, in_c), lambda i: (i, 0)),
                  pl.BlockSpec((in_c, kc), lambda i: (0, 0))],
        out_specs=pl.BlockSpec((blk, kc), lambda i: (i, 0)),
        out_shape=jax.ShapeDtypeStruct((n, kc), jnp.float32),
    )(x, wcat)


def _stats_body(x_ref, o_ref):
    @pl.when(pl.program_id(0) == 0)
    def _():
        o_ref[...] = jnp.zeros_like(o_ref)

    x = x_ref[...]
    o_ref[...] += jnp.stack([jnp.sum(x, axis=0), jnp.sum(x * x, axis=0)])


def _stats(x_pad, n, blk):
    c = x_pad.shape[1]
    return pl.pallas_call(
        _stats_body,
        grid=(n // blk,),
        in_specs=[pl.BlockSpec((blk, c), lambda i: (i, 0))],
        out_specs=pl.BlockSpec((2, c), lambda i: (0, 0)),
        out_shape=jax.ShapeDtypeStruct((2, c), jnp.float32),
    )(x_pad)


def _apply_body(x_ref, sc_ref, sh_ref, skip_ref, o_ref):
    o_ref[...] = x_ref[...] * sc_ref[0, :] + sh_ref[0, :] + skip_ref[...]


def _bn_apply(x_pad, scale, shift, skip, blk):
    n, c = skip.shape
    return pl.pallas_call(
        _apply_body,
        grid=(n // blk,),
        in_specs=[pl.BlockSpec((blk, c), lambda i: (i, 0)),
                  pl.BlockSpec((1, c), lambda i: (0, 0)),
                  pl.BlockSpec((1, c), lambda i: (0, 0)),
                  pl.BlockSpec((blk, c), lambda i: (i, 0))],
        out_specs=pl.BlockSpec((blk, c), lambda i: (i, 0)),
        out_shape=jax.ShapeDtypeStruct((n, c), jnp.float32),
    )(x_pad, scale, shift, skip)


def _make_sc_scatter(n_src_rows, n_out, out_c, Pp, C, Wn, G, R):
    NCH = -(-n_out // C)
    NP = -(-NCH // NCORES)
    S = Pp // NTILES
    nwin = S // Wn
    NW = K * nwin
    assert NW % 2 == 0
    GV = 16
    GH = 8
    ngrp = Wn // (LANES * GV)
    nblk = C // (NTILES * R)
    acc_rows = C + LANES
    n_out_pad = NCH * C

    mesh = plsc.VectorSubcoreMesh(core_axis_name="c", subcore_axis_name="s")

    @functools.partial(
        pl.kernel,
        out_type=jax.ShapeDtypeStruct((n_out_pad, out_c), jnp.float32),
        mesh=mesh,
        scratch_types=[
            pltpu.VMEM((2, Wn), jnp.int32),
            pltpu.VMEM((2, Wn), jnp.int32),
            pltpu.VMEM((G,), jnp.int32),
            pltpu.VMEM((G,), jnp.int32),
            pltpu.VMEM((G, out_c), jnp.float32),
            pltpu.VMEM((R, out_c), jnp.float32),
            pltpu.VMEM_SHARED((acc_rows, out_c), jnp.float32),
            pltpu.SemaphoreType.DMA,
            pltpu.SemaphoreType.DMA,
            pltpu.SemaphoreType.DMA,
        ],
        compiler_params=pltpu.CompilerParams(needs_layout_passes=False,
                                             use_tc_tiling_on_sc=False),
    )
    def sck(y_hbm, c_hbm, z_hbm, out_hbm,
            bufA, bufB, src_st, dst_st, rows, zbuf, acc, sem, sem2, sem3):
        cid = lax.axis_index("c")
        sid = lax.axis_index("s")
        lane = lax.iota(jnp.int32, LANES)
        dummy_src = lane
        dummy_dst = C + lane
        uC = jnp.uint32(C)

        pltpu.sync_copy(z_hbm, zbuf)

        if True:
            def flush():
                copy = pltpu.async_copy(y_hbm.at[src_st], rows, sem3)
                copy.wait()
                pltpu.sync_copy(rows, acc.at[dst_st], add=True)

            def pass_body(p, _):
                chunk = NCORES * p + cid
                valid = chunk < NCH
                base = chunk * C

                @pl.when(valid)
                def _zero():
                    for b in range(nblk):
                        r0 = (sid * nblk + b) * R
                        pltpu.sync_copy(zbuf, acc.at[pl.ds(r0, R)])

                plsc.subcore_barrier()

                def slab(widx):
                    kk = widx // nwin
                    ww = widx % nwin
                    return kk * (NTILES * nwin) + sid * nwin + ww

                def process(buf, widx, off_v):
                    kk = widx // nwin

                    def gbody(gi, off_v):
                        off_s = jnp.max(off_v)
                        do_flush = off_s > G - GV * LANES

                        @pl.when(do_flush)
                        def _():
                            for t in range(GV):
                                fi = off_s + t * LANES + lane
                                fm = fi < G
                                plsc.store_scatter(src_st, [fi],
                                                   dummy_src, mask=fm)
                                plsc.store_scatter(dst_st, [fi],
                                                   dummy_dst, mask=fm)
                            flush()

                        off_v = jnp.where(do_flush,
                                          jnp.zeros_like(off_v), off_v)
                        for h in range(GV // GH):
                            w = []
                            for t in range(GH):
                                j = gi * GV + h * GH + t
                                ov = buf[0, pl.ds(j * LANES, LANES)]
                                iv = buf[1, pl.ds(j * LANES, LANES)]
                                rel = ov - base
                                m = rel.astype(jnp.uint32) < uC
                                mi = m.astype(jnp.int32)
                                cs = plsc.cumsum(mi) - mi
                                pc = plsc.all_reduce_population_count(m)
                                w.append((rel, m, cs, pc, iv * K + kk))
                            for rel, m, cs, pc, src in w:
                                pos = off_v + cs
                                plsc.store_scatter(dst_st, [pos], rel,
                                                   mask=m)
                                plsc.store_scatter(src_st, [pos], src,
                                                   mask=m)
                                off_v = off_v + pc
                        return off_v

                    return lax.fori_loop(0, ngrp, gbody, off_v)

                pltpu.async_copy(c_hbm.at[slab(0)], bufA, sem)
                pltpu.async_copy(c_hbm.at[slab(1)], bufB, sem2)

                def wpair_body(i, off_v):
                    w0 = 2 * i
                    pltpu.make_async_copy(c_hbm.at[slab(w0)],
                                          bufA, sem).wait()
                    off_v = process(bufA, w0, off_v)

                    @pl.when(w0 + 2 < NW)
                    def _():
                        pltpu.async_copy(c_hbm.at[slab(w0 + 2)], bufA, sem)

                    pltpu.make_async_copy(c_hbm.at[slab(w0 + 1)],
                                          bufB, sem2).wait()
                    off_v = process(bufB, w0 + 1, off_v)

                    @pl.when(w0 + 3 < NW)
                    def _():
                        pltpu.async_copy(c_hbm.at[slab(w0 + 3)], bufB, sem2)

                    return off_v

                off_v = lax.fori_loop(0, NW // 2, wpair_body,
                                      jnp.zeros((LANES,), jnp.int32))

                @pl.when(valid)
                def _final():
                    off_s = jnp.max(off_v)

                    def fill(i, _):
                        fi = off_s + i * LANES + lane
                        fm = fi < G
                        plsc.store_scatter(src_st, [fi], dummy_src, mask=fm)
                        plsc.store_scatter(dst_st, [fi], dummy_dst, mask=fm)
                        return _

                    lax.fori_loop(0, G // LANES, fill, jnp.int32(0))
                    flush()

                plsc.subcore_barrier()

                @pl.when(valid)
                def _writeout():
                    for b in range(nblk):
                        r0 = (sid * nblk + b) * R
                        pltpu.sync_copy(acc.at[pl.ds(r0, R)],
                                        rows.at[pl.ds(0, R)])
                        pltpu.sync_copy(rows.at[pl.ds(0, R)],
                                        out_hbm.at[pl.ds(base + r0, R)])

                plsc.subcore_barrier()
                return _

            lax.fori_loop(0, NP, pass_body, jnp.int32(0))

    return sck, n_out_pad


def _pad_pairs(idx, Pp, fill):
    pad = Pp - idx.shape[1]
    return jnp.pad(idx.astype(jnp.int32), ((0, 0), (0, pad)),
                   constant_values=fill)


def _stage(x, W, g, b, skip, in_idx, out_idx, cfg):
    n_out = skip.shape[0]
    in_c = x.shape[1]
    out_c = W.shape[-1]
    C, Wn, G, R, Pp, gemm_blk, ew_blk = cfg

    wcat = jnp.transpose(W, (1, 0, 2)).reshape(in_c, K * out_c)
    y = _gemm(x, wcat, gemm_blk).reshape(-1, out_c)

    sck, n_out_pad = _make_sc_scatter(y.shape[0], n_out, out_c, Pp, C, Wn, G, R)
    oi = _pad_pairs(out_idx, Pp, n_out_pad + C)
    ii = _pad_pairs(in_idx, Pp, 0)
    nwin = Pp // (NTILES * Wn)
    comb = jnp.stack([oi.reshape(K, NTILES, nwin, Wn),
                      ii.reshape(K, NTILES, nwin, Wn)], axis=3)
    comb = comb.reshape(K * NTILES * nwin, 2, Wn)
    zblk = jnp.zeros((R, out_c), jnp.float32)
    out_pre = sck(y, comb, zblk)

    st = _stats(out_pre, n_out, ew_blk)
    mean = st[0] / n_out
    var = st[1] / n_out - mean * mean
    scale = (g / jnp.sqrt(var + EPS)).reshape(1, out_c)
    shift = (b - mean * scale[0]).reshape(1, out_c)
    return _bn_apply(out_pre, scale, shift, skip, ew_blk)


def kernel(x_feat, skip3, skip2, skip1, W3, g3, b3, W2, g2, b2, W1, g1, b1,
           in_idx3, out_idx3, in_idx2, out_idx2, in_idx1, out_idx1):
    cfg3 = (1600, 2048, 1024, 100, 65536,  1000, 1000)
    cfg2 = (6400, 6656, 1024, 400, 212992, 1000, 1000)
    cfg1 = (12800, 6400, 1024, 800, 409600, 1000, 1000)
    x = _stage(x_feat, W3, g3, b3, skip3, in_idx3, out_idx3, cfg3)
    x = _stage(x, W2, g2, b2, skip2, in_idx2, out_idx2, cfg2)
    x = _stage(x, W1, g1, b1, skip1, in_idx1, out_idx1, cfg1)
    return x


# speedup vs baseline: 1.6445x; 1.0122x over previous
import functools

import jax
import jax.numpy as jnp
from jax import lax
from jax.experimental import pallas as pl
from jax.experimental.pallas import tpu as pltpu
from jax.experimental.pallas import tpu_sc as plsc

K = 9
EPS = 1e-5
LANES = 16
NTILES = 16
NCORES = 2


def _gemm_body(x_ref, w_ref, o_ref):
    o_ref[...] = jnp.dot(x_ref[...], w_ref[...],
                         preferred_element_type=jnp.float32)


def _gemm(x, wcat, blk):
    n, in_c = x.shape
    kc = wcat.shape[1]
    return pl.pallas_call(
        _gemm_body,
        grid=(n // blk,),
        in_specs=[pl.BlockSpec((blk, in_c), lambda i: (i, 0)),
                  pl.BlockSpec((in_c, kc), lambda i: (0, 0))],
        out_specs=pl.BlockSpec((blk, kc), lambda i: (i, 0)),
        out_shape=jax.ShapeDtypeStruct((n, kc), jnp.float32),
    )(x, wcat)


def _stats_body(x_ref, o_ref):
    @pl.when(pl.program_id(0) == 0)
    def _():
        o_ref[...] = jnp.zeros_like(o_ref)

    x = x_ref[...]
    o_ref[...] += jnp.stack([jnp.sum(x, axis=0), jnp.sum(x * x, axis=0)])


def _stats(x_pad, n, blk):
    c = x_pad.shape[1]
    return pl.pallas_call(
        _stats_body,
        grid=(n // blk,),
        in_specs=[pl.BlockSpec((blk, c), lambda i: (i, 0))],
        out_specs=pl.BlockSpec((2, c), lambda i: (0, 0)),
        out_shape=jax.ShapeDtypeStruct((2, c), jnp.float32),
    )(x_pad)


def _apply_body(x_ref, sc_ref, sh_ref, skip_ref, o_ref):
    o_ref[...] = x_ref[...] * sc_ref[0, :] + sh_ref[0, :] + skip_ref[...]


def _bn_apply(x_pad, scale, shift, skip, blk):
    n, c = skip.shape
    return pl.pallas_call(
        _apply_body,
        grid=(n // blk,),
        in_specs=[pl.BlockSpec((blk, c), lambda i: (i, 0)),
                  pl.BlockSpec((1, c), lambda i: (0, 0)),
                  pl.BlockSpec((1, c), lambda i: (0, 0)),
                  pl.BlockSpec((blk, c), lambda i: (i, 0))],
        out_specs=pl.BlockSpec((blk, c), lambda i: (i, 0)),
        out_shape=jax.ShapeDtypeStruct((n, c), jnp.float32),
    )(x_pad, scale, shift, skip)


def _make_sc_scatter(n_src_rows, n_out, out_c, Pp, C, Wn, G, R):
    NCH = -(-n_out // C)
    NP = -(-NCH // NCORES)
    S = Pp // NTILES
    nwin = S // Wn
    NW = K * nwin
    assert NW % 2 == 0
    GV = 16
    GH = 8
    ngrp = Wn // (LANES * GV)
    nblk = C // (NTILES * R)
    acc_rows = C + LANES
    n_out_pad = NCH * C

    mesh = plsc.VectorSubcoreMesh(core_axis_name="c", subcore_axis_name="s")

    @functools.partial(
        pl.kernel,
        out_type=jax.ShapeDtypeStruct((n_out_pad, out_c), jnp.float32),
        mesh=mesh,
        scratch_types=[
            pltpu.VMEM((2, Wn), jnp.int32),
            pltpu.VMEM((2, Wn), jnp.int32),
            pltpu.VMEM((G,), jnp.int32),
            pltpu.VMEM((G,), jnp.int32),
            pltpu.VMEM((G, out_c), jnp.float32),
            pltpu.VMEM((R, out_c), jnp.float32),
            pltpu.VMEM_SHARED((acc_rows, out_c), jnp.float32),
            pltpu.SemaphoreType.DMA,
            pltpu.SemaphoreType.DMA,
            pltpu.SemaphoreType.DMA,
        ],
        compiler_params=pltpu.CompilerParams(needs_layout_passes=False,
                                             use_tc_tiling_on_sc=False,
                                             internal_scratch_in_bytes=65536),
    )
    def sck(y_hbm, c_hbm, z_hbm, out_hbm,
            bufA, bufB, src_st, dst_st, rows, zbuf, acc, sem, sem2, sem3):
        cid = lax.axis_index("c")
        sid = lax.axis_index("s")
        lane = lax.iota(jnp.int32, LANES)
        dummy_src = lane
        dummy_dst = C + lane
        uC = jnp.uint32(C)

        pltpu.sync_copy(z_hbm, zbuf)

        if True:
            def flush():
                copy = pltpu.async_copy(y_hbm.at[src_st], rows, sem3)
                copy.wait()
                pltpu.sync_copy(rows, acc.at[dst_st], add=True)

            def pass_body(p, _):
                chunk = NCORES * p + cid
                valid = chunk < NCH
                base = chunk * C

                @pl.when(valid)
                def _zero():
                    for b in range(nblk):
                        r0 = (sid * nblk + b) * R
                        pltpu.sync_copy(zbuf, acc.at[pl.ds(r0, R)])

                plsc.subcore_barrier()

                def slab(widx):
                    kk = widx // nwin
                    ww = widx % nwin
                    return kk * (NTILES * nwin) + sid * nwin + ww

                def process(buf, widx, off_v):
                    kk = widx // nwin

                    def gbody(gi, off_v):
                        off_s = jnp.max(off_v)
                        do_flush = off_s > G - GV * LANES

                        @pl.when(do_flush)
                        def _():
                            for t in range(GV):
                                fi = off_s + t * LANES + lane
                                fm = fi < G
                                plsc.store_scatter(src_st, [fi],
                                                   dummy_src, mask=fm)
                                plsc.store_scatter(dst_st, [fi],
                                                   dummy_dst, mask=fm)
                            flush()

                        off_v = jnp.where(do_flush,
                                          jnp.zeros_like(off_v), off_v)
                        for h in range(GV // GH):
                            w = []
                            for t in range(GH):
                                j = gi * GV + h * GH + t
                                ov = buf[0, pl.ds(j * LANES, LANES)]
                                iv = buf[1, pl.ds(j * LANES, LANES)]
                                rel = ov - base
                                m = rel.astype(jnp.uint32) < uC
                                mi = m.astype(jnp.int32)
                                cs = plsc.cumsum(mi) - mi
                                pc = plsc.all_reduce_population_count(m)
                                w.append((rel, m, cs, pc, iv * K + kk))
                            for rel, m, cs, pc, src in w:
                                pos = off_v + cs
                                plsc.store_scatter(dst_st, [pos], rel,
                                                   mask=m)
                                plsc.store_scatter(src_st, [pos], src,
                                                   mask=m)
                                off_v = off_v + pc
                        return off_v

                    return lax.fori_loop(0, ngrp, gbody, off_v)

                pltpu.async_copy(c_hbm.at[slab(0)], bufA, sem)
                pltpu.async_copy(c_hbm.at[slab(1)], bufB, sem2)

                def wpair_body(i, off_v):
                    w0 = 2 * i
                    pltpu.make_async_copy(c_hbm.at[slab(w0)],
                                          bufA, sem).wait()
                    off_v = process(bufA, w0, off_v)

                    @pl.when(w0 + 2 < NW)
                    def _():
                        pltpu.async_copy(c_hbm.at[slab(w0 + 2)], bufA, sem)

                    pltpu.make_async_copy(c_hbm.at[slab(w0 + 1)],
                                          bufB, sem2).wait()
                    off_v = process(bufB, w0 + 1, off_v)

                    @pl.when(w0 + 3 < NW)
                    def _():
                        pltpu.async_copy(c_hbm.at[slab(w0 + 3)], bufB, sem2)

                    return off_v

                off_v = lax.fori_loop(0, NW // 2, wpair_body,
                                      jnp.zeros((LANES,), jnp.int32))

                @pl.when(valid)
                def _final():
                    off_s = jnp.max(off_v)

                    def fill(i, _):
                        fi = off_s + i * LANES + lane
                        fm = fi < G
                        plsc.store_scatter(src_st, [fi], dummy_src, mask=fm)
                        plsc.store_scatter(dst_st, [fi], dummy_dst, mask=fm)
                        return _

                    lax.fori_loop(0, G // LANES, fill, jnp.int32(0))
                    flush()

                plsc.subcore_barrier()

                @pl.when(valid)
                def _writeout():
                    for b in range(nblk):
                        r0 = (sid * nblk + b) * R
                        pltpu.sync_copy(acc.at[pl.ds(r0, R)],
                                        rows.at[pl.ds(0, R)])
                        pltpu.sync_copy(rows.at[pl.ds(0, R)],
                                        out_hbm.at[pl.ds(base + r0, R)])

                plsc.subcore_barrier()
                return _

            lax.fori_loop(0, NP, pass_body, jnp.int32(0))

    return sck, n_out_pad


def _pad_pairs(idx, Pp, fill):
    pad = Pp - idx.shape[1]
    return jnp.pad(idx.astype(jnp.int32), ((0, 0), (0, pad)),
                   constant_values=fill)


def _stage(x, W, g, b, skip, in_idx, out_idx, cfg):
    n_out = skip.shape[0]
    in_c = x.shape[1]
    out_c = W.shape[-1]
    C, Wn, G, R, Pp, gemm_blk, ew_blk = cfg

    wcat = jnp.transpose(W, (1, 0, 2)).reshape(in_c, K * out_c)
    y = _gemm(x, wcat, gemm_blk).reshape(-1, out_c)

    sck, n_out_pad = _make_sc_scatter(y.shape[0], n_out, out_c, Pp, C, Wn, G, R)
    oi = _pad_pairs(out_idx, Pp, n_out_pad + C)
    ii = _pad_pairs(in_idx, Pp, 0)
    nwin = Pp // (NTILES * Wn)
    comb = jnp.stack([oi.reshape(K, NTILES, nwin, Wn),
                      ii.reshape(K, NTILES, nwin, Wn)], axis=3)
    comb = comb.reshape(K * NTILES * nwin, 2, Wn)
    zblk = jnp.zeros((R, out_c), jnp.float32)
    out_pre = sck(y, comb, zblk)

    st = _stats(out_pre, n_out, ew_blk)
    mean = st[0] / n_out
    var = st[1] / n_out - mean * mean
    scale = (g / jnp.sqrt(var + EPS)).reshape(1, out_c)
    shift = (b - mean * scale[0]).reshape(1, out_c)
    return _bn_apply(out_pre, scale, shift, skip, ew_blk)


def kernel(x_feat, skip3, skip2, skip1, W3, g3, b3, W2, g2, b2, W1, g1, b1,
           in_idx3, out_idx3, in_idx2, out_idx2, in_idx1, out_idx1):
    cfg3 = (2240, 2048, 1024, 140, 65536,  1000, 1000)
    cfg2 = (8800, 6656, 1024, 550, 212992, 1000, 1000)
    cfg1 = (17600, 6400, 1024, 550, 409600, 1000, 1000)
    x = _stage(x_feat, W3, g3, b3, skip3, in_idx3, out_idx3, cfg3)
    x = _stage(x, W2, g2, b2, skip2, in_idx2, out_idx2, cfg2)
    x = _stage(x, W1, g1, b1, skip1, in_idx1, out_idx1, cfg1)
    return x
